# baseline (speedup 1.0000x reference)
import functools

import jax
import jax.numpy as jnp
from jax import lax
from jax.experimental import pallas as pl
from jax.experimental.pallas import tpu as pltpu
from jax.experimental.pallas import tpu_sc as plsc

N = 4096
D = 128
NC = 64
KEEP = N - int(0.3 * N)
RES = 3584
NBUF = 1
BROWS = RES // NBUF
CH = 896
KPB = BROWS // CH
NRC = RES // CH
TAIL = N - RES
TB = 64
NTT = TAIL // TB
NSTEPS = 3
NSC = 8
SCORE_CHUNK = N // NSC

_ALPHA = 0.5


def _res_copy(b, k, adj_hbm, bufs, rsems):
    return pltpu.make_async_copy(
        adj_hbm.at[pl.ds(b * BROWS + k * CH, CH)],
        bufs[b].at[pl.ds(k * CH, CH)],
        rsems.at[k * NBUF + b])


def _tail_copy(t, adj_hbm, tbuf, tsems):
    j = t % NTT
    return pltpu.make_async_copy(
        adj_hbm.at[pl.ds(RES + j * TB, TB)],
        tbuf.at[t % 2],
        tsems.at[t % 2])


def _tc_body(he_ref, hc_ref, adj_hbm, w1_ref, pp_ref,
             mask_ref, posd_ref, ar0, tbuf, h_row, a_row,
             rsems, tsems):
    bufs = (ar0,)
    for k in range(KPB):
        for b in range(NBUF):
            _res_copy(b, k, adj_hbm, bufs, rsems).start()
    for t in range(2):
        _tail_copy(t, adj_hbm, tbuf, tsems).start()

    hc = hc_ref[...]
    w1 = w1_ref[...]
    for c in range(NSC):
        he = he_ref[pl.ds(c * SCORE_CHUNK, SCORE_CHUNK), :]
        score1 = jnp.dot(he, w1)
        logits = lax.dot_general(he, hc, (((1,), (1,)), ((), ())))
        sw = jax.nn.softmax(logits, axis=-1)
        wc = jnp.dot(sw, hc)
        rel = jnp.sum(wc, axis=1, keepdims=True)
        score = _ALPHA * score1 + (1.0 - _ALPHA) * (-jnp.log(rel + 1e-10))
        h_row[0:1, pl.ds(c * SCORE_CHUNK, SCORE_CHUNK)] = jnp.reshape(
            score, (1, SCORE_CHUNK))

    def _gru():
        w_msg = pp_ref[0, 0]
        b_msg = pp_ref[0, 1]
        w_z = pp_ref[0, 2]
        u_z = pp_ref[0, 3]
        b_z = pp_ref[0, 4]
        w_r = pp_ref[0, 5]
        u_r = pp_ref[0, 6]
        b_r = pp_ref[0, 7]
        w_h = pp_ref[0, 8]
        u_h = pp_ref[0, 9]
        b_h = pp_ref[0, 10]
        hv = h_row[...]
        a = a_row[...] * w_msg + b_msg
        z = jax.nn.sigmoid(a * w_z + hv * u_z + b_z)
        r = jax.nn.sigmoid(a * w_r + hv * u_r + b_r)
        h_t = jnp.tanh(a * w_h + (r * hv) * u_h + b_h)
        h_row[...] = (1.0 - z) * hv + z * h_t

    for k in range(KPB):
        for b in range(NBUF):
            _res_copy(b, k, adj_hbm, bufs, rsems).wait()
            part = lax.dot_general(
                h_row[...], bufs[b][pl.ds(k * CH, CH), :],
                (((1,), (1,)), ((), ())))
            a_row[0:1, pl.ds(b * BROWS + k * CH, CH)] = part
    for j in range(NTT):
        t = j
        _tail_copy(t, adj_hbm, tbuf, tsems).wait()
        part = lax.dot_general(
            h_row[...], tbuf[t % 2], (((1,), (1,)), ((), ())))
        a_row[0:1, pl.ds(RES + j * TB, TB)] = part
        if t + 2 < NSTEPS * NTT:
            _tail_copy(t + 2, adj_hbm, tbuf, tsems).start()
    _gru()

    for s in range(1, NSTEPS):
        for b in range(NBUF):
            a_row[0:1, pl.ds(b * BROWS, BROWS)] = lax.dot_general(
                h_row[...], bufs[b][...], (((1,), (1,)), ((), ())))
        for j in range(NTT):
            t = s * NTT + j
            _tail_copy(t, adj_hbm, tbuf, tsems).wait()
            part = lax.dot_general(
                h_row[...], tbuf[t % 2], (((1,), (1,)), ((), ())))
            a_row[0:1, pl.ds(RES + j * TB, TB)] = part
            if t + 2 < NSTEPS * NTT:
                _tail_copy(t + 2, adj_hbm, tbuf, tsems).start()
        _gru()

    sc = h_row[...]
    sc = jnp.where(sc == 0.0, 0.0, sc)
    mat = jnp.reshape(sc, (N // 128, 128))
    bits = pltpu.bitcast(mat, jnp.int32)
    sgn = bits >> 31
    key = bits ^ (sgn & jnp.int32(0x7FFFFFFF))

    def _bisect(_, carry):
        lo, hi = carry
        mid = (lo & hi) + ((lo ^ hi) >> 1)
        cnt = jnp.sum((key <= mid).astype(jnp.int32))
        take = cnt >= KEEP
        return (jnp.where(take, lo, mid + 1), jnp.where(take, mid, hi))

    lo0 = jnp.int32(-2147483647 - 1)
    hi0 = jnp.int32(2147483647)
    _, thr = lax.fori_loop(0, 32, _bisect, (lo0, hi0))

    cnt_lt = jnp.sum((key < thr).astype(jnp.int32))
    need = (KEEP - cnt_lt).astype(jnp.float32)
    eq = (key == thr).astype(jnp.float32)
    li = lax.broadcasted_iota(jnp.int32, (128, 128), 0)
    ci = lax.broadcasted_iota(jnp.int32, (128, 128), 1)
    ut = (li < ci).astype(jnp.float32)
    p_in = jnp.dot(eq, ut)
    row_tot = jnp.sum(eq, axis=1, keepdims=True)
    ri = lax.broadcasted_iota(jnp.int32, (N // 128, N // 128), 0)
    rj = lax.broadcasted_iota(jnp.int32, (N // 128, N // 128), 1)
    lt = (rj < ri).astype(jnp.float32)
    row_off = jnp.dot(lt, row_tot)
    tie_pre = p_in + row_off
    keep = (key < thr) | ((key == thr) & (tie_pre < need))
    mask_ref[...] = keep.astype(jnp.int32)
    keepf = keep.astype(jnp.float32)
    kp_in = jnp.dot(keepf, ut)
    krow_tot = jnp.sum(keepf, axis=1, keepdims=True)
    krow_off = jnp.dot(lt, krow_tot)
    kpi = (kp_in + krow_off).astype(jnp.int32)
    lin = (lax.broadcasted_iota(jnp.int32, (N // 128, 128), 0) * 128
           + lax.broadcasted_iota(jnp.int32, (N // 128, 128), 1))
    posd_ref[...] = jnp.where(keep, kpi, KEEP + lin - kpi)


def _tc_scores_mask(H_e, H_c, adj_e, W_score1, params):
    return pl.pallas_call(
        _tc_body,
        in_specs=[
            pl.BlockSpec((N, D), lambda: (0, 0)),
            pl.BlockSpec((NC, D), lambda: (0, 0)),
            pl.BlockSpec(memory_space=pl.ANY),
            pl.BlockSpec((D, 1), lambda: (0, 0)),
            pl.BlockSpec((1, 11), lambda: (0, 0), memory_space=pltpu.SMEM),
        ],
        out_specs=[pl.BlockSpec((N // 128, 128), lambda: (0, 0)),
                   pl.BlockSpec((N // 128, 128), lambda: (0, 0))],
        out_shape=[jax.ShapeDtypeStruct((N // 128, 128), jnp.int32),
                   jax.ShapeDtypeStruct((N // 128, 128), jnp.int32)],
        scratch_shapes=[
            pltpu.VMEM((BROWS, N), jnp.float32),
            pltpu.VMEM((2, TB, N), jnp.float32),
            pltpu.VMEM((1, N), jnp.float32),
            pltpu.VMEM((1, N), jnp.float32),
            pltpu.SemaphoreType.DMA((NRC,)),
            pltpu.SemaphoreType.DMA((2,)),
        ],
        compiler_params=pltpu.CompilerParams(
            vmem_limit_bytes=100 * 1024 * 1024),
    )(H_e, H_c, adj_e, W_score1, params)


_WPN = 32
_CHW = N // _WPN


def _sc_body(posd_hbm, he_hbm, out_hbm, pos_vmem, rows_vmem, sem):
    cidx = lax.axis_index("c")
    sidx = lax.axis_index("s")
    wid = sidx * 2 + cidx
    base = wid * _CHW
    pltpu.sync_copy(posd_hbm.at[pl.ds(base, _CHW)], pos_vmem)
    pltpu.sync_copy(he_hbm.at[pl.ds(base, _CHW)], rows_vmem)
    pltpu.async_copy(rows_vmem, out_hbm.at[pos_vmem], sem).wait()


def _sc_scatter(posd_flat, H_e):
    mesh = plsc.VectorSubcoreMesh(core_axis_name="c", subcore_axis_name="s")
    kfn = functools.partial(
        pl.kernel,
        mesh=mesh,
        out_type=jax.ShapeDtypeStruct((N, D), jnp.float32),
        scratch_types=[
            pltpu.VMEM((_CHW,), jnp.int32),
            pltpu.VMEM((_CHW, D), jnp.float32),
            pltpu.SemaphoreType.DMA,
        ],
    )(_sc_body)
    return kfn(posd_flat, H_e)


def kernel(H_e, H_c, adj_e, W_score1, W_msg, b_msg, W_z, U_z, b_z,
           W_r, U_r, b_r, W_h, U_h, b_h):
    params = jnp.concatenate([
        W_msg.reshape(-1), b_msg.reshape(-1), W_z.reshape(-1),
        U_z.reshape(-1), b_z.reshape(-1), W_r.reshape(-1),
        U_r.reshape(-1), b_r.reshape(-1), W_h.reshape(-1),
        U_h.reshape(-1), b_h.reshape(-1),
    ]).reshape(1, 11)
    mask_mat, posd_mat = _tc_scores_mask(H_e, H_c, adj_e, W_score1, params)
    keep_mask = mask_mat.reshape(N) != 0
    refined = _sc_scatter(posd_mat.reshape(N), H_e)[:KEEP]
    return (refined, keep_mask)


# speedup vs baseline: 1.0168x; 1.0168x over previous
import functools

import jax
import jax.numpy as jnp
from jax import lax
from jax.experimental import pallas as pl
from jax.experimental.pallas import tpu as pltpu
from jax.experimental.pallas import tpu_sc as plsc

N = 4096
D = 128
NC = 64
KEEP = N - int(0.3 * N)
RES = 3584
NBUF = 2
BROWS = RES // NBUF
CH = 448
KPB = BROWS // CH
NRC = RES // CH
TAIL = N - RES
TB = 64
NTT = TAIL // TB
NSTEPS = 3
NSC = 8
SCORE_CHUNK = N // NSC

_ALPHA = 0.5


def _res_copy(b, k, adj_hbm, bufs, rsems):
    return pltpu.make_async_copy(
        adj_hbm.at[pl.ds(b * BROWS + k * CH, CH)],
        bufs[b].at[pl.ds(k * CH, CH)],
        rsems.at[k * NBUF + b])


def _tail_copy(t, adj_hbm, tbuf, tsems):
    j = t % NTT
    return pltpu.make_async_copy(
        adj_hbm.at[pl.ds(RES + j * TB, TB)],
        tbuf.at[t % 2],
        tsems.at[t % 2])


def _tc_body(he_ref, hc_ref, adj_hbm, w1_ref, pp_ref,
             mask_ref, posd_ref, ar0, ar1, tbuf, h_row, a_row,
             rsems, tsems):
    bufs = (ar0, ar1)
    for k in range(KPB):
        for b in range(NBUF):
            _res_copy(b, k, adj_hbm, bufs, rsems).start()
    for t in range(2):
        _tail_copy(t, adj_hbm, tbuf, tsems).start()

    hc = hc_ref[...]
    w1 = w1_ref[...]
    for c in range(NSC):
        he = he_ref[pl.ds(c * SCORE_CHUNK, SCORE_CHUNK), :]
        score1 = jnp.dot(he, w1)
        logits = lax.dot_general(he, hc, (((1,), (1,)), ((), ())))
        sw = jax.nn.softmax(logits, axis=-1)
        wc = jnp.dot(sw, hc)
        rel = jnp.sum(wc, axis=1, keepdims=True)
        score = _ALPHA * score1 + (1.0 - _ALPHA) * (-jnp.log(rel + 1e-10))
        h_row[0:1, pl.ds(c * SCORE_CHUNK, SCORE_CHUNK)] = jnp.reshape(
            score, (1, SCORE_CHUNK))

    def _gru():
        w_msg = pp_ref[0, 0]
        b_msg = pp_ref[0, 1]
        w_z = pp_ref[0, 2]
        u_z = pp_ref[0, 3]
        b_z = pp_ref[0, 4]
        w_r = pp_ref[0, 5]
        u_r = pp_ref[0, 6]
        b_r = pp_ref[0, 7]
        w_h = pp_ref[0, 8]
        u_h = pp_ref[0, 9]
        b_h = pp_ref[0, 10]
        hv = h_row[...]
        a = a_row[...] * w_msg + b_msg
        z = jax.nn.sigmoid(a * w_z + hv * u_z + b_z)
        r = jax.nn.sigmoid(a * w_r + hv * u_r + b_r)
        h_t = jnp.tanh(a * w_h + (r * hv) * u_h + b_h)
        h_row[...] = (1.0 - z) * hv + z * h_t

    for k in range(KPB):
        for b in range(NBUF):
            _res_copy(b, k, adj_hbm, bufs, rsems).wait()
            part = lax.dot_general(
                h_row[...], bufs[b][pl.ds(k * CH, CH), :],
                (((1,), (1,)), ((), ())))
            a_row[0:1, pl.ds(b * BROWS + k * CH, CH)] = part
    for j in range(NTT):
        t = j
        _tail_copy(t, adj_hbm, tbuf, tsems).wait()
        part = lax.dot_general(
            h_row[...], tbuf[t % 2], (((1,), (1,)), ((), ())))
        a_row[0:1, pl.ds(RES + j * TB, TB)] = part
        if t + 2 < NSTEPS * NTT:
            _tail_copy(t + 2, adj_hbm, tbuf, tsems).start()
    _gru()

    for s in range(1, NSTEPS):
        for b in range(NBUF):
            a_row[0:1, pl.ds(b * BROWS, BROWS)] = lax.dot_general(
                h_row[...], bufs[b][...], (((1,), (1,)), ((), ())))
        for j in range(NTT):
            t = s * NTT + j
            _tail_copy(t, adj_hbm, tbuf, tsems).wait()
            part = lax.dot_general(
                h_row[...], tbuf[t % 2], (((1,), (1,)), ((), ())))
            a_row[0:1, pl.ds(RES + j * TB, TB)] = part
            if t + 2 < NSTEPS * NTT:
                _tail_copy(t + 2, adj_hbm, tbuf, tsems).start()
        _gru()

    sc = h_row[...]
    sc = jnp.where(sc == 0.0, 0.0, sc)
    mat = jnp.reshape(sc, (N // 128, 128))
    bits = pltpu.bitcast(mat, jnp.int32)
    sgn = bits >> 31
    key = bits ^ (sgn & jnp.int32(0x7FFFFFFF))

    def _bisect(_, carry):
        lo, hi = carry
        mid = (lo & hi) + ((lo ^ hi) >> 1)
        cnt = jnp.sum((key <= mid).astype(jnp.int32))
        take = cnt >= KEEP
        return (jnp.where(take, lo, mid + 1), jnp.where(take, mid, hi))

    lo0 = jnp.int32(-2147483647 - 1)
    hi0 = jnp.int32(2147483647)
    _, thr = lax.fori_loop(0, 32, _bisect, (lo0, hi0))

    cnt_lt = jnp.sum((key < thr).astype(jnp.int32))
    need = (KEEP - cnt_lt).astype(jnp.float32)
    eq = (key == thr).astype(jnp.float32)
    li = lax.broadcasted_iota(jnp.int32, (128, 128), 0)
    ci = lax.broadcasted_iota(jnp.int32, (128, 128), 1)
    ut = (li < ci).astype(jnp.float32)
    p_in = jnp.dot(eq, ut)
    row_tot = jnp.sum(eq, axis=1, keepdims=True)
    ri = lax.broadcasted_iota(jnp.int32, (N // 128, N // 128), 0)
    rj = lax.broadcasted_iota(jnp.int32, (N // 128, N // 128), 1)
    lt = (rj < ri).astype(jnp.float32)
    row_off = jnp.dot(lt, row_tot)
    tie_pre = p_in + row_off
    keep = (key < thr) | ((key == thr) & (tie_pre < need))
    mask_ref[...] = keep.astype(jnp.int32)
    keepf = keep.astype(jnp.float32)
    kp_in = jnp.dot(keepf, ut)
    krow_tot = jnp.sum(keepf, axis=1, keepdims=True)
    krow_off = jnp.dot(lt, krow_tot)
    kpi = (kp_in + krow_off).astype(jnp.int32)
    lin = (lax.broadcasted_iota(jnp.int32, (N // 128, 128), 0) * 128
           + lax.broadcasted_iota(jnp.int32, (N // 128, 128), 1))
    posd_ref[...] = jnp.where(keep, kpi, KEEP + lin - kpi)


def _tc_scores_mask(H_e, H_c, adj_e, W_score1, params):
    return pl.pallas_call(
        _tc_body,
        in_specs=[
            pl.BlockSpec((N, D), lambda: (0, 0)),
            pl.BlockSpec((NC, D), lambda: (0, 0)),
            pl.BlockSpec(memory_space=pl.ANY),
            pl.BlockSpec((D, 1), lambda: (0, 0)),
            pl.BlockSpec((1, 11), lambda: (0, 0), memory_space=pltpu.SMEM),
        ],
        out_specs=[pl.BlockSpec((N // 128, 128), lambda: (0, 0)),
                   pl.BlockSpec((N // 128, 128), lambda: (0, 0))],
        out_shape=[jax.ShapeDtypeStruct((N // 128, 128), jnp.int32),
                   jax.ShapeDtypeStruct((N // 128, 128), jnp.int32)],
        scratch_shapes=[
            pltpu.VMEM((BROWS, N), jnp.float32),
            pltpu.VMEM((BROWS, N), jnp.float32),
            pltpu.VMEM((2, TB, N), jnp.float32),
            pltpu.VMEM((1, N), jnp.float32),
            pltpu.VMEM((1, N), jnp.float32),
            pltpu.SemaphoreType.DMA((NRC,)),
            pltpu.SemaphoreType.DMA((2,)),
        ],
        compiler_params=pltpu.CompilerParams(
            vmem_limit_bytes=100 * 1024 * 1024),
    )(H_e, H_c, adj_e, W_score1, params)


_WPN = 32
_CHW = N // _WPN


def _sc_body(posd_hbm, he_hbm, out_hbm, pos_vmem, rows_vmem, sem):
    cidx = lax.axis_index("c")
    sidx = lax.axis_index("s")
    wid = sidx * 2 + cidx
    base = wid * _CHW
    pltpu.sync_copy(posd_hbm.at[pl.ds(base, _CHW)], pos_vmem)
    pltpu.sync_copy(he_hbm.at[pl.ds(base, _CHW)], rows_vmem)
    pltpu.async_copy(rows_vmem, out_hbm.at[pos_vmem], sem).wait()


def _sc_scatter(posd_flat, H_e):
    mesh = plsc.VectorSubcoreMesh(core_axis_name="c", subcore_axis_name="s")
    kfn = functools.partial(
        pl.kernel,
        mesh=mesh,
        out_type=jax.ShapeDtypeStruct((N, D), jnp.float32),
        scratch_types=[
            pltpu.VMEM((_CHW,), jnp.int32),
            pltpu.VMEM((_CHW, D), jnp.float32),
            pltpu.SemaphoreType.DMA,
        ],
    )(_sc_body)
    return kfn(posd_flat, H_e)


def kernel(H_e, H_c, adj_e, W_score1, W_msg, b_msg, W_z, U_z, b_z,
           W_r, U_r, b_r, W_h, U_h, b_h):
    params = jnp.concatenate([
        W_msg.reshape(-1), b_msg.reshape(-1), W_z.reshape(-1),
        U_z.reshape(-1), b_z.reshape(-1), W_r.reshape(-1),
        U_r.reshape(-1), b_r.reshape(-1), W_h.reshape(-1),
        U_h.reshape(-1), b_h.reshape(-1),
    ]).reshape(1, 11)
    mask_mat, posd_mat = _tc_scores_mask(H_e, H_c, adj_e, W_score1, params)
    keep_mask = mask_mat.reshape(N) != 0
    refined = _sc_scatter(posd_mat.reshape(N), H_e)[:KEEP]
    return (refined, keep_mask)
